# baseline (speedup 1.0000x reference)
import jax
import jax.numpy as jnp
from jax.experimental import pallas as pl


def _add_kernel(x_ref, pos_ref, out_ref):
    out_ref[...] = x_ref[...] + pos_ref[...][None, :, :]


def kernel(x, pos_table):
    b, s, e = x.shape
    return pl.pallas_call(
        _add_kernel,
        grid=(b,),
        in_specs=[
            pl.BlockSpec((1, s, e), lambda i: (i, 0, 0)),
            pl.BlockSpec((s, e), lambda i: (0, 0)),
        ],
        out_specs=pl.BlockSpec((1, s, e), lambda i: (i, 0, 0)),
        out_shape=jax.ShapeDtypeStruct((b, s, e), x.dtype),
    )(x, pos_table)
